# baseline (speedup 1.0000x reference)
import functools

import jax
import jax.numpy as jnp
from jax import lax
from jax.experimental import pallas as pl
from jax.experimental.pallas import tpu as pltpu
from jax.experimental.pallas import tpu_sc as plsc

BATCH = 4
SEQ = 4096
DIM = 1024

_NC = 2
_NS = 16
_NW = _NC * _NS

_CH_ROWS = 16
_CH = _CH_ROWS * DIM
_ROWS_PER_W = SEQ // _NW
_NCHUNK = _ROWS_PER_W // _CH_ROWS


def _make_sc_add():
    mesh = plsc.VectorSubcoreMesh(core_axis_name="c", subcore_axis_name="s")

    @functools.partial(
        pl.kernel,
        mesh=mesh,
        out_type=jax.ShapeDtypeStruct((BATCH, SEQ * DIM), jnp.float32),
        scratch_types=[
            pltpu.VMEM((_CH,), jnp.float32),
            pltpu.VMEM((_CH,), jnp.float32),
            pltpu.VMEM((_CH,), jnp.float32),
            pltpu.SemaphoreType.DMA,
            pltpu.SemaphoreType.DMA,
            pltpu.SemaphoreType.DMA,
            pltpu.SemaphoreType.DMA,
        ],
    )
    def sc_add(in_hbm, pos_hbm, out_hbm, pos_v, io0, io1, si0, si1, so0, so1):
        wid = lax.axis_index("s") * _NC + lax.axis_index("c")
        base = wid * _ROWS_PER_W * DIM

        io = (io0, io1)
        sin = (si0, si1)
        sout = (so0, so1)
        steps = [(ci, b) for ci in range(_NCHUNK) for b in range(BATCH)]
        nst = len(steps)

        def in_load(t):
            ci, b = steps[t]
            off = base + ci * _CH
            return pltpu.async_copy(in_hbm.at[b, pl.ds(off, _CH)], io[t % 2], sin[t % 2])

        load_h = {0: in_load(0)}
        store_h = {}

        for t in range(nst):
            ci, b = steps[t]
            off = base + ci * _CH
            buf = t % 2
            if t + 1 < nst:
                if t >= 1:
                    store_h[t - 1].wait()
                load_h[t + 1] = in_load(t + 1)
            if b == 0:
                pltpu.sync_copy(pos_hbm.at[pl.ds(off, _CH)], pos_v)
            load_h[t].wait()
            io_ref = io[buf]

            @plsc.parallel_loop(0, _CH // 16, unroll=8)
            def add_body(i):
                s = pl.ds(i * 16, 16)
                io_ref[s] = io_ref[s] + pos_v[s]
            store_h[t] = pltpu.async_copy(io_ref, out_hbm.at[b, pl.ds(off, _CH)], sout[buf])

        store_h[nst - 2].wait()
        store_h[nst - 1].wait()

    return sc_add


_sc_add = _make_sc_add()


def kernel(inputs, pos_table):
    batch, seq, dim = inputs.shape
    out = _sc_add(inputs.reshape(batch, seq * dim), pos_table.reshape(seq * dim))
    return out.reshape(batch, seq, dim)


# speedup vs baseline: 2.1209x; 2.1209x over previous
import functools

import jax
import jax.numpy as jnp
from jax import lax
from jax.experimental import pallas as pl
from jax.experimental.pallas import tpu as pltpu
from jax.experimental.pallas import tpu_sc as plsc

BATCH = 4
SEQ = 4096
DIM = 1024

_NC = 2
_NS = 16
_NW = _NC * _NS

_CH_ROWS = 16
_ROWS_PER_W = SEQ // _NW
_NCHUNK = _ROWS_PER_W // _CH_ROWS


def _make_sc_add():
    mesh = plsc.VectorSubcoreMesh(core_axis_name="c", subcore_axis_name="s")

    @functools.partial(
        pl.kernel,
        mesh=mesh,
        out_type=jax.ShapeDtypeStruct((BATCH * SEQ, DIM), jnp.float32),
        compiler_params=pltpu.CompilerParams(use_tc_tiling_on_sc=True),
        scratch_types=[
            pltpu.VMEM((_CH_ROWS, DIM), jnp.float32),
            pltpu.VMEM((_CH_ROWS, DIM), jnp.float32),
            pltpu.VMEM((_CH_ROWS, DIM), jnp.float32),
            pltpu.SemaphoreType.DMA,
            pltpu.SemaphoreType.DMA,
            pltpu.SemaphoreType.DMA,
            pltpu.SemaphoreType.DMA,
        ],
    )
    def sc_add(in_hbm, pos_hbm, out_hbm, pos_v, io0, io1, si0, si1, so0, so1):
        wid = lax.axis_index("s") * _NC + lax.axis_index("c")
        row0 = wid * _ROWS_PER_W

        io = (io0, io1)
        sin = (si0, si1)
        sout = (so0, so1)
        steps = [(ci, b) for ci in range(_NCHUNK) for b in range(BATCH)]
        nst = len(steps)

        def in_load(t):
            ci, b = steps[t]
            r = b * SEQ + row0 + ci * _CH_ROWS
            return pltpu.async_copy(
                in_hbm.at[pl.ds(r, _CH_ROWS), :], io[t % 2], sin[t % 2]
            )

        load_h = {0: in_load(0)}
        store_h = {}

        for t in range(nst):
            ci, b = steps[t]
            r = b * SEQ + row0 + ci * _CH_ROWS
            buf = t % 2
            if t + 1 < nst:
                if t >= 1:
                    store_h[t - 1].wait()
                load_h[t + 1] = in_load(t + 1)
            if b == 0:
                pltpu.sync_copy(
                    pos_hbm.at[pl.ds(row0 + ci * _CH_ROWS, _CH_ROWS), :], pos_v
                )
            load_h[t].wait()
            io_ref = io[buf]

            def add_row(rr, _):
                @plsc.parallel_loop(0, DIM // 16, unroll=8)
                def add_col(c):
                    s = pl.ds(c * 16, 16)
                    io_ref[rr, s] = io_ref[rr, s] + pos_v[rr, s]

                return 0

            lax.fori_loop(0, _CH_ROWS, add_row, 0)
            store_h[t] = pltpu.async_copy(
                io_ref, out_hbm.at[pl.ds(r, _CH_ROWS), :], sout[buf]
            )

        store_h[nst - 2].wait()
        store_h[nst - 1].wait()

    return sc_add


_sc_add = _make_sc_add()


def kernel(inputs, pos_table):
    batch, seq, dim = inputs.shape
    out = _sc_add(inputs.reshape(batch * seq, dim), pos_table)
    return out.reshape(batch, seq, dim)
